# baseline (speedup 1.0000x reference)
import math

import jax
import jax.numpy as jnp
from jax import lax
from jax.experimental import pallas as pl

SCALE = 1.0 / 0.6


def _ssilu(x):
    return x * jax.nn.sigmoid(x) * SCALE


def _dot(a, b):
    return jnp.dot(a, b, preferred_element_type=jnp.float32)


def _dot_t(a, b):
    return lax.dot_general(a, b, (((0,), (0,)), ((), ())),
                           preferred_element_type=jnp.float32)


def _main_body(F, B, M,
               batchc_ref, batchr_ref, as_ref, av_ref, ef_ref, dir_ref,
               axcat_ref,
               mpW1_ref, mpb1_ref, mpW2_ref, mpb2_ref, rW_ref, rb_ref,
               sW1_ref, sb1_ref, sW2_ref, sb2_ref, vW_ref,
               nas_ref, nav_ref, sacc_ref):
    i = pl.program_id(0)
    inv3 = 1.0 / math.sqrt(3.0)
    invh = 1.0 / math.sqrt(F)

    @pl.when(i == 0)
    def _init():
        sacc_ref[...] = jnp.zeros_like(sacc_ref)

    as_ = as_ref[...]
    h = _ssilu(_dot(as_, mpW1_ref[...]) + mpb1_ref[...])
    ap = _dot(h, mpW2_ref[...]) + mpb2_ref[...]
    ep = _dot(ef_ref[...], rW_ref[...]) + rb_ref[...]
    m = ap * ep * inv3
    m1 = m[:, :F] * invh
    m2 = m[:, F:2 * F] * invh
    as2 = m[:, 2 * F:] + as_

    bcol = batchc_ref[0]
    brow = batchr_ref[0]
    oh = (bcol == lax.broadcasted_iota(jnp.int32, (M, B), 1)
          ).astype(jnp.float32)
    ohT = (brow == lax.broadcasted_iota(jnp.int32, (B, M), 0)
           ).astype(jnp.float32)

    G = _dot(oh, axcat_ref[...])
    gs = G[:, :F]
    t = _ssilu(_dot(as2, sW1_ref[:F]) + _dot(gs, sW1_ref[F:]) + sb1_ref[...])
    nas = _ssilu(_dot(t, sW2_ref[...]) + sb2_ref[...]) + as2
    nas_ref[...] = nas

    vW = vW_ref[...]
    nvs = [nas]
    for a in range(3):
        av_a = av_ref[:, a, :]
        av2_a = m1 * av_a + m2 * dir_ref[:, a:a + 1]
        gv_a = G[:, (1 + a) * F:(2 + a) * F]
        nv_a = _dot(av2_a + gv_a, vW) + av2_a
        nav_ref[:, a, :] = nv_a
        nvs.append(nv_a)
    nvs.append(jnp.ones((M, 8), jnp.float32))
    sacc_ref[...] += _dot(ohT, jnp.concatenate(nvs, axis=1))


def _tail_body(F, B,
               sacc_ref, axs_ref, axv_ref,
               aW1_ref, ab1_ref, aW2_ref, ab2_ref, tvW_ref,
               svW_ref, ssW1_ref, ssb1_ref, ssW2_ref, ssb2_ref, ldw_ref,
               outs_ref, outv_ref, outd_ref):
    sacc = sacc_ref[...]
    r = 1.0 / jnp.maximum(sacc[:, 4 * F:4 * F + 1], 1.0)
    rF = jnp.broadcast_to(r, (B, F))
    axs = axs_ref[...]

    pooled_s = sacc[:, :F] * rF
    t = _ssilu(_dot(pooled_s, aW1_ref[:F]) + _dot(axs, aW1_ref[F:])
               + ab1_ref[...])
    ds = _ssilu(_dot(t, aW2_ref[...]) + ab2_ref[...])
    ass = axs + ds
    outs_parts = []
    v1 = []
    v2 = []
    avs = []
    tvW = tvW_ref[...]
    svW = svW_ref[...]
    for a in range(3):
        axv_a = axv_ref[:, a, :]
        pooled_v_a = sacc[:, (1 + a) * F:(2 + a) * F] * rF
        avs_a = axv_a + _dot(pooled_v_a + axv_a, tvW)
        avs.append(avs_a)
        w12 = _dot(avs_a, svW)
        v1.append(w12[:, :F])
        v2.append(w12[:, F:])
    vnorm = jnp.sqrt(v2[0] * v2[0] + v2[1] * v2[1] + v2[2] * v2[2] + 1e-8)
    sh = _dot(_ssilu(_dot(ass, ssW1_ref[:F]) + _dot(vnorm, ssW1_ref[F:])
                     + ssb1_ref[...]), ssW2_ref[...]) + ssb2_ref[...]
    s1 = sh[:, :F]
    s2 = sh[:, F:2 * F]
    gate = jnp.tanh(sh[:, 2 * F:])
    outs_ref[...] = s2 + ass * gate
    ldw = ldw_ref[...]
    deltas = []
    for a in range(3):
        ov_a = s1 * v1[a] + avs[a]
        outv_ref[:, a, :] = ov_a
        deltas.append(jnp.sum(ov_a * ldw, axis=1, keepdims=True))
    outd_ref[...] = jnp.concatenate(deltas, axis=1)


def kernel(atom_scalar, axis_scalar_state, atom_vector, axis_vector_state,
           axis_edge_feat, axis_edge_dir, batch, params):
    N, F = atom_scalar.shape
    B = axis_scalar_state.shape[0]
    R = axis_edge_feat.shape[1]
    p = params

    M = 1000
    assert N % M == 0
    nblk = N // M
    batch3 = batch.reshape(nblk, 1, M)
    batchc = batch.reshape(nblk, M, 1)

    row = lambda b: b.reshape(1, -1)
    import functools
    main = pl.pallas_call(
        functools.partial(_main_body, F, B, M),
        grid=(nblk,),
        in_specs=[
            pl.BlockSpec((1, M, 1), lambda i: (i, 0, 0)),
            pl.BlockSpec((1, 1, M), lambda i: (i, 0, 0)),
            pl.BlockSpec((M, F), lambda i: (i, 0)),
            pl.BlockSpec((M, 3, F), lambda i: (i, 0, 0)),
            pl.BlockSpec((M, R), lambda i: (i, 0)),
            pl.BlockSpec((M, 3), lambda i: (i, 0)),
            pl.BlockSpec((B, 4 * F), lambda i: (0, 0)),
            pl.BlockSpec((F, F), lambda i: (0, 0)),
            pl.BlockSpec((1, F), lambda i: (0, 0)),
            pl.BlockSpec((F, 3 * F), lambda i: (0, 0)),
            pl.BlockSpec((1, 3 * F), lambda i: (0, 0)),
            pl.BlockSpec((R, 3 * F), lambda i: (0, 0)),
            pl.BlockSpec((1, 3 * F), lambda i: (0, 0)),
            pl.BlockSpec((2 * F, F), lambda i: (0, 0)),
            pl.BlockSpec((1, F), lambda i: (0, 0)),
            pl.BlockSpec((F, F), lambda i: (0, 0)),
            pl.BlockSpec((1, F), lambda i: (0, 0)),
            pl.BlockSpec((F, F), lambda i: (0, 0)),
        ],
        out_specs=[
            pl.BlockSpec((M, F), lambda i: (i, 0)),
            pl.BlockSpec((M, 3, F), lambda i: (i, 0, 0)),
            pl.BlockSpec((B, 4 * F + 8), lambda i: (0, 0)),
        ],
        out_shape=[
            jax.ShapeDtypeStruct((N, F), jnp.float32),
            jax.ShapeDtypeStruct((N, 3, F), jnp.float32),
            jax.ShapeDtypeStruct((B, 4 * F + 8), jnp.float32),
        ],
    )
    axcat = jnp.concatenate(
        [axis_scalar_state, axis_vector_state.reshape(B, 3 * F)], axis=1)
    nas, nav, sacc = main(
        batchc, batch3, atom_scalar, atom_vector, axis_edge_feat, axis_edge_dir,
        axcat,
        p['mp_W1'], row(p['mp_b1']), p['mp_W2'], row(p['mp_b2']),
        p['rbf_W'], row(p['rbf_b']),
        p['a2a_s_W1'], row(p['a2a_s_b1']), p['a2a_s_W2'], row(p['a2a_s_b2']),
        p['a2a_v_W'])

    tail = pl.pallas_call(
        functools.partial(_tail_body, F, B),
        out_shape=[
            jax.ShapeDtypeStruct((B, F), jnp.float32),
            jax.ShapeDtypeStruct((B, 3, F), jnp.float32),
            jax.ShapeDtypeStruct((B, 3), jnp.float32),
        ],
    )
    outs, outv, outd = tail(
        sacc, axis_scalar_state, axis_vector_state,
        p['ats_W1'], row(p['ats_b1']), p['ats_W2'], row(p['ats_b2']),
        p['atv_W'], p['sv_W'],
        p['ss_W1'], row(p['ss_b1']), p['ss_W2'], row(p['ss_b2']),
        p['ld_W'].reshape(1, F))

    return nas, nav, outs, outv, outd[:, :, None]


# speedup vs baseline: 1.0365x; 1.0365x over previous
import math

import jax
import jax.numpy as jnp
from jax import lax
from jax.experimental import pallas as pl

SCALE = 1.0 / 0.6


def _ssilu(x):
    return x * jax.nn.sigmoid(x) * SCALE


def _dot(a, b):
    return jnp.dot(a, b, preferred_element_type=jnp.float32)


def _dot_t(a, b):
    return lax.dot_general(a, b, (((0,), (0,)), ((), ())),
                           preferred_element_type=jnp.float32)


def _main_body(F, B, M,
               batch_ref, as_ref, av_ref, ef_ref, dir_ref, axs_ref, axv_ref,
               mpW1_ref, mpb1_ref, mpW2_ref, mpb2_ref, rW_ref, rb_ref,
               sW1_ref, sb1_ref, sW2_ref, sb2_ref, vW_ref, e3_ref,
               nas_ref, nav_ref, ps_ref, pv_ref, cnt_ref):
    i = pl.program_id(0)
    inv3 = 1.0 / math.sqrt(3.0)
    invh = 1.0 / math.sqrt(F)

    @pl.when(i == 0)
    def _init():
        ps_ref[...] = jnp.zeros_like(ps_ref)
        pv_ref[...] = jnp.zeros_like(pv_ref)
        cnt_ref[...] = jnp.zeros_like(cnt_ref)

    as_ = as_ref[...]
    h = _ssilu(_dot(as_, mpW1_ref[...]) + mpb1_ref[...])
    ap = _dot(h, mpW2_ref[...]) + mpb2_ref[...]
    ep = _dot(ef_ref[...], rW_ref[...]) + rb_ref[...]
    m = ap * ep * inv3
    m1 = m[:, :F] * invh
    m2 = m[:, F:2 * F] * invh
    as2 = m[:, 2 * F:] + as_

    bidx = batch_ref[0, 0, :]
    oh = (bidx[:, None] == lax.broadcasted_iota(jnp.int32, (M, B), 1)
          ).astype(jnp.float32)

    gs = _dot(oh, axs_ref[...])
    t = _ssilu(_dot(as2, sW1_ref[:F]) + _dot(gs, sW1_ref[F:]) + sb1_ref[...])
    nas = _ssilu(_dot(t, sW2_ref[...]) + sb2_ref[...]) + as2
    nas_ref[...] = nas
    ps_ref[...] += _dot_t(oh, nas)
    cnt_ref[...] += _dot_t(oh, jnp.ones((M, 8), jnp.float32))

    vW = vW_ref[...]
    dbc = _dot(dir_ref[...], e3_ref[...])
    for a in range(3):
        av_a = av_ref[:, a, :]
        av2_a = m1 * av_a + m2 * dbc[:, a * F:(a + 1) * F]
        gv_a = _dot(oh, axv_ref[:, a, :])
        nv_a = _dot(av2_a + gv_a, vW) + av2_a
        nav_ref[:, a, :] = nv_a
        pv_ref[a] += _dot_t(oh, nv_a)


def _tail_body(F, B,
               ps_ref, pv_ref, cnt_ref, axs_ref, axv_ref,
               aW1_ref, ab1_ref, aW2_ref, ab2_ref, tvW_ref,
               svW_ref, ssW1_ref, ssb1_ref, ssW2_ref, ssb2_ref, ldw_ref,
               outs_ref, outv_ref, outd_ref):
    r = 1.0 / jnp.maximum(cnt_ref[...], 1.0)
    rF = jnp.broadcast_to(r[:, 0:1], (B, F))
    axs = axs_ref[...]

    pooled_s = ps_ref[...] * rF
    t = _ssilu(_dot(pooled_s, aW1_ref[:F]) + _dot(axs, aW1_ref[F:])
               + ab1_ref[...])
    ds = _ssilu(_dot(t, aW2_ref[...]) + ab2_ref[...])
    ass = axs + ds
    outs_parts = []
    v1 = []
    v2 = []
    avs = []
    tvW = tvW_ref[...]
    svW = svW_ref[...]
    for a in range(3):
        axv_a = axv_ref[:, a, :]
        pooled_v_a = pv_ref[a] * rF
        avs_a = axv_a + _dot(pooled_v_a + axv_a, tvW)
        avs.append(avs_a)
        w12 = _dot(avs_a, svW)
        v1.append(w12[:, :F])
        v2.append(w12[:, F:])
    vnorm = jnp.sqrt(v2[0] * v2[0] + v2[1] * v2[1] + v2[2] * v2[2] + 1e-8)
    sh = _dot(_ssilu(_dot(ass, ssW1_ref[:F]) + _dot(vnorm, ssW1_ref[F:])
                     + ssb1_ref[...]), ssW2_ref[...]) + ssb2_ref[...]
    s1 = sh[:, :F]
    s2 = sh[:, F:2 * F]
    gate = jnp.tanh(sh[:, 2 * F:])
    outs_ref[...] = s2 + ass * gate
    ldw = ldw_ref[...]
    deltas = []
    for a in range(3):
        ov_a = s1 * v1[a] + avs[a]
        outv_ref[:, a, :] = ov_a
        deltas.append(jnp.sum(ov_a * ldw, axis=1, keepdims=True))
    outd_ref[...] = jnp.concatenate(deltas, axis=1)


def kernel(atom_scalar, axis_scalar_state, atom_vector, axis_vector_state,
           axis_edge_feat, axis_edge_dir, batch, params):
    N, F = atom_scalar.shape
    B = axis_scalar_state.shape[0]
    R = axis_edge_feat.shape[1]
    p = params

    M = 1000
    assert N % M == 0
    nblk = N // M
    batch3 = batch.reshape(nblk, 1, M)

    row = lambda b: b.reshape(1, -1)
    import functools
    main = pl.pallas_call(
        functools.partial(_main_body, F, B, M),
        grid=(nblk,),
        in_specs=[
            pl.BlockSpec((1, 1, M), lambda i: (i, 0, 0)),
            pl.BlockSpec((M, F), lambda i: (i, 0)),
            pl.BlockSpec((M, 3, F), lambda i: (i, 0, 0)),
            pl.BlockSpec((M, R), lambda i: (i, 0)),
            pl.BlockSpec((M, 3), lambda i: (i, 0)),
            pl.BlockSpec((B, F), lambda i: (0, 0)),
            pl.BlockSpec((B, 3, F), lambda i: (0, 0, 0)),
            pl.BlockSpec((F, F), lambda i: (0, 0)),
            pl.BlockSpec((1, F), lambda i: (0, 0)),
            pl.BlockSpec((F, 3 * F), lambda i: (0, 0)),
            pl.BlockSpec((1, 3 * F), lambda i: (0, 0)),
            pl.BlockSpec((R, 3 * F), lambda i: (0, 0)),
            pl.BlockSpec((1, 3 * F), lambda i: (0, 0)),
            pl.BlockSpec((2 * F, F), lambda i: (0, 0)),
            pl.BlockSpec((1, F), lambda i: (0, 0)),
            pl.BlockSpec((F, F), lambda i: (0, 0)),
            pl.BlockSpec((1, F), lambda i: (0, 0)),
            pl.BlockSpec((F, F), lambda i: (0, 0)),
            pl.BlockSpec((3, 3 * F), lambda i: (0, 0)),
        ],
        out_specs=[
            pl.BlockSpec((M, F), lambda i: (i, 0)),
            pl.BlockSpec((M, 3, F), lambda i: (i, 0, 0)),
            pl.BlockSpec((B, F), lambda i: (0, 0)),
            pl.BlockSpec((3, B, F), lambda i: (0, 0, 0)),
            pl.BlockSpec((B, 8), lambda i: (0, 0)),
        ],
        out_shape=[
            jax.ShapeDtypeStruct((N, F), jnp.float32),
            jax.ShapeDtypeStruct((N, 3, F), jnp.float32),
            jax.ShapeDtypeStruct((B, F), jnp.float32),
            jax.ShapeDtypeStruct((3, B, F), jnp.float32),
            jax.ShapeDtypeStruct((B, 8), jnp.float32),
        ],
    )
    nas, nav, ps, pv, cnt = main(
        batch3, atom_scalar, atom_vector, axis_edge_feat, axis_edge_dir,
        axis_scalar_state, axis_vector_state,
        p['mp_W1'], row(p['mp_b1']), p['mp_W2'], row(p['mp_b2']),
        p['rbf_W'], row(p['rbf_b']),
        p['a2a_s_W1'], row(p['a2a_s_b1']), p['a2a_s_W2'], row(p['a2a_s_b2']),
        p['a2a_v_W'],
        jnp.kron(jnp.eye(3, dtype=jnp.float32), jnp.ones((1, F), jnp.float32)))

    tail = pl.pallas_call(
        functools.partial(_tail_body, F, B),
        out_shape=[
            jax.ShapeDtypeStruct((B, F), jnp.float32),
            jax.ShapeDtypeStruct((B, 3, F), jnp.float32),
            jax.ShapeDtypeStruct((B, 3), jnp.float32),
        ],
    )
    outs, outv, outd = tail(
        ps, pv, cnt, axis_scalar_state, axis_vector_state,
        p['ats_W1'], row(p['ats_b1']), p['ats_W2'], row(p['ats_b2']),
        p['atv_W'], p['sv_W'],
        p['ss_W1'], row(p['ss_b1']), p['ss_W2'], row(p['ss_b2']),
        p['ld_W'].reshape(1, F))

    return nas, nav, outs, outv, outd[:, :, None]


# speedup vs baseline: 1.0534x; 1.0163x over previous
import math

import jax
import jax.numpy as jnp
from jax import lax
from jax.experimental import pallas as pl

SCALE = 1.0 / 0.6


def _ssilu(x):
    return x * jax.nn.sigmoid(x) * SCALE


def _dot(a, b):
    return jnp.dot(a, b, preferred_element_type=jnp.float32)


def _dot_t(a, b):
    return lax.dot_general(a, b, (((0,), (0,)), ((), ())),
                           preferred_element_type=jnp.float32)


def _main_body(F, B, M,
               batch_ref, as_ref, av_ref, ef_ref, dir_ref, axs_ref, axv_ref,
               mpW1_ref, mpb1_ref, mpW2_ref, mpb2_ref, rW_ref, rb_ref,
               sW1_ref, sb1_ref, sW2_ref, sb2_ref, vW_ref,
               nas_ref, nav_ref, ps_ref, pv_ref, cnt_ref):
    i = pl.program_id(0)
    inv3 = 1.0 / math.sqrt(3.0)
    invh = 1.0 / math.sqrt(F)

    @pl.when(i == 0)
    def _init():
        ps_ref[...] = jnp.zeros_like(ps_ref)
        pv_ref[...] = jnp.zeros_like(pv_ref)
        cnt_ref[...] = jnp.zeros_like(cnt_ref)

    as_ = as_ref[...]
    h = _ssilu(_dot(as_, mpW1_ref[...]) + mpb1_ref[...])
    ap = _dot(h, mpW2_ref[...]) + mpb2_ref[...]
    ep = _dot(ef_ref[...], rW_ref[...]) + rb_ref[...]
    m = ap * ep * inv3
    m1 = m[:, :F] * invh
    m2 = m[:, F:2 * F] * invh
    as2 = m[:, 2 * F:] + as_

    bidx = batch_ref[0, 0, :]
    oh = (bidx[:, None] == lax.broadcasted_iota(jnp.int32, (M, B), 1)
          ).astype(jnp.float32)

    gs = _dot(oh, axs_ref[...])
    t = _ssilu(_dot(as2, sW1_ref[:F]) + _dot(gs, sW1_ref[F:]) + sb1_ref[...])
    nas = _ssilu(_dot(t, sW2_ref[...]) + sb2_ref[...]) + as2
    nas_ref[...] = nas
    ps_ref[...] += _dot_t(oh, nas)
    cnt_ref[...] += _dot_t(oh, jnp.ones((M, 8), jnp.float32))

    vW = vW_ref[...]
    for a in range(3):
        av_a = av_ref[:, a, :]
        av2_a = m1 * av_a + m2 * dir_ref[:, a:a + 1]
        gv_a = _dot(oh, axv_ref[:, a, :])
        nv_a = _dot(av2_a + gv_a, vW) + av2_a
        nav_ref[:, a, :] = nv_a
        pv_ref[a] += _dot_t(oh, nv_a)


def _tail_body(F, B,
               ps_ref, pv_ref, cnt_ref, axs_ref, axv_ref,
               aW1_ref, ab1_ref, aW2_ref, ab2_ref, tvW_ref,
               svW_ref, ssW1_ref, ssb1_ref, ssW2_ref, ssb2_ref, ldw_ref,
               outs_ref, outv_ref, outd_ref):
    r = 1.0 / jnp.maximum(cnt_ref[...], 1.0)
    rF = jnp.broadcast_to(r[:, 0:1], (B, F))
    axs = axs_ref[...]

    pooled_s = ps_ref[...] * rF
    t = _ssilu(_dot(pooled_s, aW1_ref[:F]) + _dot(axs, aW1_ref[F:])
               + ab1_ref[...])
    ds = _ssilu(_dot(t, aW2_ref[...]) + ab2_ref[...])
    ass = axs + ds
    outs_parts = []
    v1 = []
    v2 = []
    avs = []
    tvW = tvW_ref[...]
    svW = svW_ref[...]
    for a in range(3):
        axv_a = axv_ref[:, a, :]
        pooled_v_a = pv_ref[a] * rF
        avs_a = axv_a + _dot(pooled_v_a + axv_a, tvW)
        avs.append(avs_a)
        w12 = _dot(avs_a, svW)
        v1.append(w12[:, :F])
        v2.append(w12[:, F:])
    vnorm = jnp.sqrt(v2[0] * v2[0] + v2[1] * v2[1] + v2[2] * v2[2] + 1e-8)
    sh = _dot(_ssilu(_dot(ass, ssW1_ref[:F]) + _dot(vnorm, ssW1_ref[F:])
                     + ssb1_ref[...]), ssW2_ref[...]) + ssb2_ref[...]
    s1 = sh[:, :F]
    s2 = sh[:, F:2 * F]
    gate = jnp.tanh(sh[:, 2 * F:])
    outs_ref[...] = s2 + ass * gate
    ldw = ldw_ref[...]
    deltas = []
    for a in range(3):
        ov_a = s1 * v1[a] + avs[a]
        outv_ref[:, a, :] = ov_a
        deltas.append(jnp.sum(ov_a * ldw, axis=1, keepdims=True))
    outd_ref[...] = jnp.concatenate(deltas, axis=1)


def kernel(atom_scalar, axis_scalar_state, atom_vector, axis_vector_state,
           axis_edge_feat, axis_edge_dir, batch, params):
    N, F = atom_scalar.shape
    B = axis_scalar_state.shape[0]
    R = axis_edge_feat.shape[1]
    p = params

    M = 1000
    assert N % M == 0
    nblk = N // M
    batch3 = batch.reshape(nblk, 1, M)

    row = lambda b: b.reshape(1, -1)
    import functools
    main = pl.pallas_call(
        functools.partial(_main_body, F, B, M),
        grid=(nblk,),
        in_specs=[
            pl.BlockSpec((1, 1, M), lambda i: (i, 0, 0)),
            pl.BlockSpec((M, F), lambda i: (i, 0)),
            pl.BlockSpec((M, 3, F), lambda i: (i, 0, 0)),
            pl.BlockSpec((M, R), lambda i: (i, 0)),
            pl.BlockSpec((M, 3), lambda i: (i, 0)),
            pl.BlockSpec((B, F), lambda i: (0, 0)),
            pl.BlockSpec((B, 3, F), lambda i: (0, 0, 0)),
            pl.BlockSpec((F, F), lambda i: (0, 0)),
            pl.BlockSpec((1, F), lambda i: (0, 0)),
            pl.BlockSpec((F, 3 * F), lambda i: (0, 0)),
            pl.BlockSpec((1, 3 * F), lambda i: (0, 0)),
            pl.BlockSpec((R, 3 * F), lambda i: (0, 0)),
            pl.BlockSpec((1, 3 * F), lambda i: (0, 0)),
            pl.BlockSpec((2 * F, F), lambda i: (0, 0)),
            pl.BlockSpec((1, F), lambda i: (0, 0)),
            pl.BlockSpec((F, F), lambda i: (0, 0)),
            pl.BlockSpec((1, F), lambda i: (0, 0)),
            pl.BlockSpec((F, F), lambda i: (0, 0)),
        ],
        out_specs=[
            pl.BlockSpec((M, F), lambda i: (i, 0)),
            pl.BlockSpec((M, 3, F), lambda i: (i, 0, 0)),
            pl.BlockSpec((B, F), lambda i: (0, 0)),
            pl.BlockSpec((3, B, F), lambda i: (0, 0, 0)),
            pl.BlockSpec((B, 8), lambda i: (0, 0)),
        ],
        out_shape=[
            jax.ShapeDtypeStruct((N, F), jnp.float32),
            jax.ShapeDtypeStruct((N, 3, F), jnp.float32),
            jax.ShapeDtypeStruct((B, F), jnp.float32),
            jax.ShapeDtypeStruct((3, B, F), jnp.float32),
            jax.ShapeDtypeStruct((B, 8), jnp.float32),
        ],
    )
    nas, nav, ps, pv, cnt = main(
        batch3, atom_scalar, atom_vector, axis_edge_feat, axis_edge_dir,
        axis_scalar_state, axis_vector_state,
        p['mp_W1'], row(p['mp_b1']), p['mp_W2'], row(p['mp_b2']),
        p['rbf_W'], row(p['rbf_b']),
        p['a2a_s_W1'], row(p['a2a_s_b1']), p['a2a_s_W2'], row(p['a2a_s_b2']),
        p['a2a_v_W'])

    tail = pl.pallas_call(
        functools.partial(_tail_body, F, B),
        out_shape=[
            jax.ShapeDtypeStruct((B, F), jnp.float32),
            jax.ShapeDtypeStruct((B, 3, F), jnp.float32),
            jax.ShapeDtypeStruct((B, 3), jnp.float32),
        ],
    )
    outs, outv, outd = tail(
        ps, pv, cnt, axis_scalar_state, axis_vector_state,
        p['ats_W1'], row(p['ats_b1']), p['ats_W2'], row(p['ats_b2']),
        p['atv_W'], p['sv_W'],
        p['ss_W1'], row(p['ss_b1']), p['ss_W2'], row(p['ss_b2']),
        p['ld_W'].reshape(1, F))

    return nas, nav, outs, outv, outd[:, :, None]
